# baseline (speedup 1.0000x reference)
import functools

import jax
import jax.numpy as jnp
from jax import lax
from jax.experimental import pallas as pl
from jax.experimental.pallas import tpu as pltpu
from jax.experimental.pallas import tpu_sc as plsc

B, H, W = 16, 512, 512
L = 16
ROW_MIN = W // 128
CHUNK_ROWS = 32
NBUF = 4
N_CHUNKS = (H // 2) // CHUNK_ROWS
PROB_T = 0.5
NPTS_T = 10
LOOSE = 64

_mesh = plsc.VectorSubcoreMesh(core_axis_name="c", subcore_axis_name="s")


_SC_SCRATCH = [
    pltpu.VMEM((NBUF, CHUNK_ROWS, W), jnp.float32),
    pltpu.VMEM((L,), jnp.int32),
    pltpu.VMEM((L,), jnp.int32),
    pltpu.VMEM((L,), jnp.int32),
] + [pltpu.SemaphoreType.DMA] * NBUF

_NJ = W // L
_HJ = _NJ // 2
_RPI = 16


def _sc_body(mask_hbm, bbox_hbm, part_hbm, chunk, meta, pmeta, bstage,
             *sems):
    c = lax.axis_index("c")
    s = lax.axis_index("s")
    b = c * 8 + s // 2
    h = s % 2
    r0 = h * (H // 2)

    io = lax.broadcasted_iota(jnp.int32, (L,), 0)
    one_i = jnp.ones((L,), jnp.int32)
    zero_i = jnp.zeros((L,), jnp.int32)

    def src(ch):
        return mask_hbm.at[b, pl.ds(r0 + ch * CHUNK_ROWS, CHUNK_ROWS)]

    colmax = [jnp.full((L,), -1.0, jnp.float32) for _ in range(_NJ)]
    rminv = jnp.full((L,), H + 1, jnp.int32)
    rmaxv = jnp.full((L,), -1, jnp.int32)
    cv0 = jnp.zeros((L,), jnp.int32)
    cv1 = jnp.zeros((L,), jnp.int32)

    handles = [None] * NBUF
    for p in range(NBUF - 1):
        handles[p] = pltpu.async_copy(src(p), chunk.at[p], sems[p])
    for ch in range(N_CHUNKS):
        buf = ch % NBUF
        nxt = ch + NBUF - 1
        if nxt < N_CHUNKS:
            nb = nxt % NBUF
            handles[nb] = pltpu.async_copy(src(nxt), chunk.at[nb], sems[nb])
        handles[buf].wait()

        for half in range(2):
            def row_body(r2, cr_, _buf=buf, _half=half):
                rminv_, rmaxv_, rsplat, cva, cvb = cr_[:5]
                cols = list(cr_[5:])
                for rr in range(_RPI):
                    ra = rb = None
                    for k in range(_HJ):
                        j = _half * _HJ + k
                        v = chunk[_buf, r2 * _RPI + rr, pl.ds(j * L, L)]
                        cols[k] = jnp.maximum(cols[k], v)
                        pc = plsc.all_reduce_population_count(v > PROB_T)
                        if k % 2 == 0:
                            cva = cva + pc
                            ra = v if ra is None else jnp.maximum(ra, v)
                        else:
                            cvb = cvb + pc
                            rb = v if rb is None else jnp.maximum(rb, v)
                    manyv = jnp.maximum(ra, rb) > PROB_T
                    rminv_ = jnp.where(manyv, jnp.minimum(rminv_, rsplat),
                                       rminv_)
                    rmaxv_ = jnp.where(manyv, jnp.maximum(rmaxv_, rsplat),
                                       rmaxv_)
                    rsplat = rsplat + one_i
                return (rminv_, rmaxv_, rsplat, cva, cvb) + tuple(cols)

            base = jnp.full((L,), ch * CHUNK_ROWS, jnp.int32)
            out = lax.fori_loop(
                0, CHUNK_ROWS // _RPI, row_body,
                (rminv, rmaxv, base, cv0, cv1)
                + tuple(colmax[half * _HJ:(half + 1) * _HJ]))
            rminv, rmaxv, _, cv0, cv1 = out[:5]
            colmax[half * _HJ:(half + 1) * _HJ] = list(out[5:])

    cnt_mine = jnp.max(cv0 + cv1)

    mn = jnp.min(rminv)
    mx = jnp.max(rmaxv)
    rmin_s = jnp.where(mn > H, jnp.int32(H + 1), mn + r0)
    rmax_s = jnp.where(mx < 0, jnp.int32(-1), mx + r0)

    cminv = jnp.full((L,), W + 1, jnp.int32)
    cmaxv = jnp.full((L,), -1, jnp.int32)
    for j in range(_NJ):
        m = colmax[j] > PROB_T
        idx = io + j * L
        cminv = jnp.minimum(cminv, jnp.where(m, idx, W + 1))
        cmaxv = jnp.maximum(cmaxv, jnp.where(m, idx, -1))
    cmin_s = jnp.min(cminv)
    cmax_s = jnp.max(cmaxv)

    wid = c * 16 + s
    mv = jnp.where(io == 0, cnt_mine,
                   jnp.where(io == 1, rmin_s,
                             jnp.where(io == 2, rmax_s,
                                       jnp.where(io == 3, cmin_s,
                                                 jnp.where(io == 4, cmax_s, 0)))))
    meta[...] = mv
    pltpu.sync_copy(meta, part_hbm.at[wid])
    plsc.subcore_barrier()

    pltpu.sync_copy(part_hbm.at[wid ^ 1], pmeta)
    pm = pmeta[...]
    cnt_tot = cnt_mine + pm[0]
    rmin = jnp.minimum(rmin_s, pm[1])
    rmax = jnp.maximum(rmax_s, pm[2])
    cmin = jnp.minimum(cmin_s, pm[3])
    cmax = jnp.maximum(cmax_s, pm[4])

    valid = cnt_tot >= NPTS_T
    rmin_f = jnp.where(valid, jnp.maximum(rmin - LOOSE, 0), 0)
    rmax_f = jnp.where(valid, jnp.minimum(rmax + LOOSE, H - 1), H - 1)
    cmin_f = jnp.where(valid, jnp.maximum(cmin - LOOSE, 0), 0)
    cmax_f = jnp.where(valid, jnp.minimum(cmax + LOOSE, W - 1), W - 1)

    bvec = jnp.where(io == 0, rmin_f,
                     jnp.where(io == 1, cmin_f,
                               jnp.where(io == 2, rmax_f,
                                         jnp.where(io == 3, cmax_f, 0))))

    @pl.when(h == 0)
    def _():
        bstage[...] = bvec
        pltpu.sync_copy(bstage, bbox_hbm.at[b])


_sc_bbox = pl.kernel(
    _sc_body,
    mesh=_mesh,
    out_type=[jax.ShapeDtypeStruct((B, 16), jnp.int32),
              jax.ShapeDtypeStruct((32, L), jnp.int32)],
    compiler_params=pltpu.CompilerParams(needs_layout_passes=False,
                                         skip_device_barrier=True),
    scratch_types=_SC_SCRATCH,
)


def _paint_body(bbox_ref, out_ref):
    b = pl.program_id(0)
    rmin = bbox_ref[b, 0]
    cmin = bbox_ref[b, 1]
    rmax = bbox_ref[b, 2]
    cmax = bbox_ref[b, 3]
    rr = lax.broadcasted_iota(jnp.int32, (1, H, W), 1)
    cc = lax.broadcasted_iota(jnp.int32, (1, H, W), 2)
    att = (rr >= rmin) & (rr <= rmax) & (cc >= cmin) & (cc <= cmax)
    out_ref[...] = att.astype(jnp.float32)


_paint = pl.pallas_call(
    _paint_body,
    grid=(B,),
    in_specs=[pl.BlockSpec(memory_space=pltpu.SMEM)],
    out_specs=pl.BlockSpec((1, H, W), lambda b: (b, 0, 0)),
    out_shape=jax.ShapeDtypeStruct((B, H, W), jnp.float32),
)


def kernel(mask):
    bbox_wide, _ = _sc_bbox(mask)
    att = _paint(bbox_wide)
    return att, bbox_wide[:, :4]


# speedup vs baseline: 1.1980x; 1.1980x over previous
import functools

import jax
import jax.numpy as jnp
from jax import lax
from jax.experimental import pallas as pl
from jax.experimental.pallas import tpu as pltpu
from jax.experimental.pallas import tpu_sc as plsc

B, H, W = 16, 512, 512
L = 16
ROW_MIN = W // 128
CHUNK_ROWS = 64
NBUF = 2
N_CHUNKS = (H // 2) // CHUNK_ROWS
PROB_T = 0.5
NPTS_T = 10
LOOSE = 64

_mesh = plsc.VectorSubcoreMesh(core_axis_name="c", subcore_axis_name="s")


_SC_SCRATCH = [
    pltpu.VMEM((NBUF, CHUNK_ROWS, W), jnp.float32),
    pltpu.VMEM((L,), jnp.int32),
    pltpu.VMEM((L,), jnp.int32),
    pltpu.VMEM((L,), jnp.int32),
    pltpu.VMEM((4, 8, W), jnp.float32),
    pltpu.SemaphoreType.DMA,
] + [pltpu.SemaphoreType.DMA] * NBUF

_NJ = W // L
_HJ = _NJ // 2
_RPI = 16


def _sc_body(mask_hbm, bbox_hbm, part_hbm, att_hbm, chunk, meta, pmeta,
             bstage, tmpl, sem_p, *sems):
    c = lax.axis_index("c")
    s = lax.axis_index("s")
    b = c * 8 + s // 2
    h = s % 2
    r0 = h * (H // 2)

    io = lax.broadcasted_iota(jnp.int32, (L,), 0)
    one_i = jnp.ones((L,), jnp.int32)
    zero_i = jnp.zeros((L,), jnp.int32)

    def src(ch):
        return mask_hbm.at[b, pl.ds(r0 + ch * CHUNK_ROWS, CHUNK_ROWS)]

    colmax = [jnp.full((L,), -1.0, jnp.float32) for _ in range(_NJ)]
    rminv = jnp.full((L,), H + 1, jnp.int32)
    rmaxv = jnp.full((L,), -1, jnp.int32)
    cv0 = jnp.zeros((L,), jnp.int32)
    cv1 = jnp.zeros((L,), jnp.int32)

    handles = [None] * NBUF
    for p in range(NBUF - 1):
        handles[p] = pltpu.async_copy(src(p), chunk.at[p], sems[p])
    for ch in range(N_CHUNKS):
        buf = ch % NBUF
        nxt = ch + NBUF - 1
        if nxt < N_CHUNKS:
            nb = nxt % NBUF
            handles[nb] = pltpu.async_copy(src(nxt), chunk.at[nb], sems[nb])
        handles[buf].wait()

        for half in range(2):
            def row_body(r2, cr_, _buf=buf, _half=half):
                rminv_, rmaxv_, rsplat, cva, cvb = cr_[:5]
                cols = list(cr_[5:])
                for rr in range(_RPI):
                    ra = rb = None
                    for k in range(_HJ):
                        j = _half * _HJ + k
                        v = chunk[_buf, r2 * _RPI + rr, pl.ds(j * L, L)]
                        cols[k] = jnp.maximum(cols[k], v)
                        pc = plsc.all_reduce_population_count(v > PROB_T)
                        if k % 2 == 0:
                            cva = cva + pc
                            ra = v if ra is None else jnp.maximum(ra, v)
                        else:
                            cvb = cvb + pc
                            rb = v if rb is None else jnp.maximum(rb, v)
                    manyv = jnp.maximum(ra, rb) > PROB_T
                    rminv_ = jnp.where(manyv, jnp.minimum(rminv_, rsplat),
                                       rminv_)
                    rmaxv_ = jnp.where(manyv, jnp.maximum(rmaxv_, rsplat),
                                       rmaxv_)
                    rsplat = rsplat + one_i
                return (rminv_, rmaxv_, rsplat, cva, cvb) + tuple(cols)

            base = jnp.full((L,), ch * CHUNK_ROWS, jnp.int32)
            out = lax.fori_loop(
                0, CHUNK_ROWS // _RPI, row_body,
                (rminv, rmaxv, base, cv0, cv1)
                + tuple(colmax[half * _HJ:(half + 1) * _HJ]))
            rminv, rmaxv, _, cv0, cv1 = out[:5]
            colmax[half * _HJ:(half + 1) * _HJ] = list(out[5:])

    cnt_mine = jnp.max(cv0 + cv1)

    mn = jnp.min(rminv)
    mx = jnp.max(rmaxv)
    rmin_s = jnp.where(mn > H, jnp.int32(H + 1), mn + r0)
    rmax_s = jnp.where(mx < 0, jnp.int32(-1), mx + r0)

    cminv = jnp.full((L,), W + 1, jnp.int32)
    cmaxv = jnp.full((L,), -1, jnp.int32)
    for j in range(_NJ):
        m = colmax[j] > PROB_T
        idx = io + j * L
        cminv = jnp.minimum(cminv, jnp.where(m, idx, W + 1))
        cmaxv = jnp.maximum(cmaxv, jnp.where(m, idx, -1))
    cmin_s = jnp.min(cminv)
    cmax_s = jnp.max(cmaxv)

    wid = c * 16 + s
    mv = jnp.where(io == 0, cnt_mine,
                   jnp.where(io == 1, rmin_s,
                             jnp.where(io == 2, rmax_s,
                                       jnp.where(io == 3, cmin_s,
                                                 jnp.where(io == 4, cmax_s, 0)))))
    meta[...] = mv
    pltpu.sync_copy(meta, part_hbm.at[wid])
    plsc.subcore_barrier()

    pltpu.sync_copy(part_hbm.at[wid ^ 1], pmeta)
    pm = pmeta[...]
    cnt_tot = cnt_mine + pm[0]
    rmin = jnp.minimum(rmin_s, pm[1])
    rmax = jnp.maximum(rmax_s, pm[2])
    cmin = jnp.minimum(cmin_s, pm[3])
    cmax = jnp.maximum(cmax_s, pm[4])

    valid = cnt_tot >= NPTS_T
    rmin_f = jnp.where(valid, jnp.maximum(rmin - LOOSE, 0), 0)
    rmax_f = jnp.where(valid, jnp.minimum(rmax + LOOSE, H - 1), H - 1)
    cmin_f = jnp.where(valid, jnp.maximum(cmin - LOOSE, 0), 0)
    cmax_f = jnp.where(valid, jnp.minimum(cmax + LOOSE, W - 1), W - 1)

    bvec = jnp.where(io == 0, rmin_f,
                     jnp.where(io == 1, cmin_f,
                               jnp.where(io == 2, rmax_f,
                                         jnp.where(io == 3, cmax_f, 0))))

    @pl.when(h == 0)
    def _():
        bstage[...] = bvec
        pltpu.sync_copy(bstage, bbox_hbm.at[b])

    zerov = jnp.zeros((L,), jnp.float32)
    onev = jnp.ones((L,), jnp.float32)
    al = rmin_f // 8
    bl = rmax_f // 8
    for j in range(_NJ):
        idx = io + j * L
        colm = (idx >= cmin_f) & (idx <= cmax_f)
        boxv = jnp.where(colm, onev, zerov)
        for i in range(8):
            tmpl[0, i, pl.ds(j * L, L)] = zerov
            tmpl[1, i, pl.ds(j * L, L)] = boxv
            ra_ = al * 8 + i
            rb_ = bl * 8 + i
            ina = (ra_ >= rmin_f) & (ra_ <= rmax_f)
            inb = (rb_ >= rmin_f) & (rb_ <= rmax_f)
            tmpl[2, i, pl.ds(j * L, L)] = jnp.where(ina, boxv, zerov)
            tmpl[3, i, pl.ds(j * L, L)] = jnp.where(inb, boxv, zerov)

    r0b = r0 // 8
    hs = []
    for w in range(32):
        aw = r0b + w
        in_mid = (aw > al) & (aw < bl)
        sel = jnp.where(aw == al, 2,
                        jnp.where(aw == bl, 3,
                                  jnp.where(in_mid, 1, 0)))
        hs.append(pltpu.async_copy(
            tmpl.at[sel], att_hbm.at[b, pl.ds(r0 + w * 8, 8)], sem_p))
        if len(hs) > 16:
            hs.pop(0).wait()
    for hh in hs:
        hh.wait()


_sc_bbox = pl.kernel(
    _sc_body,
    mesh=_mesh,
    out_type=[jax.ShapeDtypeStruct((B, 16), jnp.int32),
              jax.ShapeDtypeStruct((32, L), jnp.int32),
              jax.ShapeDtypeStruct((B, H, W), jnp.float32)],
    compiler_params=pltpu.CompilerParams(needs_layout_passes=False,
                                         skip_device_barrier=True),
    scratch_types=_SC_SCRATCH,
)


def _paint_body(bbox_ref, out_ref):
    b = pl.program_id(0)
    rmin = bbox_ref[b, 0]
    cmin = bbox_ref[b, 1]
    rmax = bbox_ref[b, 2]
    cmax = bbox_ref[b, 3]
    rr = lax.broadcasted_iota(jnp.int32, (1, H, W), 1)
    cc = lax.broadcasted_iota(jnp.int32, (1, H, W), 2)
    att = (rr >= rmin) & (rr <= rmax) & (cc >= cmin) & (cc <= cmax)
    out_ref[...] = att.astype(jnp.float32)


_paint = pl.pallas_call(
    _paint_body,
    grid=(B,),
    in_specs=[pl.BlockSpec(memory_space=pltpu.SMEM)],
    out_specs=pl.BlockSpec((1, H, W), lambda b: (b, 0, 0)),
    out_shape=jax.ShapeDtypeStruct((B, H, W), jnp.float32),
)


def kernel(mask):
    bbox_wide, _, att = _sc_bbox(mask)
    return att, bbox_wide[:, :4]
